# Initial kernel scaffold; baseline (speedup 1.0000x reference)
import jax
import jax.numpy as jnp
from jax.experimental import pallas as pl


def kernel(features, Wg, bg, W1, b1, W2, b2, Wv1, bv1, Wv2, bv2):
    raise NotImplementedError("write your pallas kernel here")


# speedup vs baseline: 1.5005x; 1.5005x over previous
import functools

import jax
import jax.numpy as jnp
from jax.experimental import pallas as pl
from jax.experimental.pallas import tpu as pltpu


def _fused_kernel(x_ref, wg_ref, bg_ref, w1_ref, b1_ref, w2_ref, b2_ref,
                  wv1_ref, bv1_ref, wv2_ref, bv2_ref, pi_ref, vf_ref,
                  *, n_exp):
    x = x_ref[...]
    t = x.shape[0]

    logits = jax.lax.dot_general(
        x, wg_ref[...], (((1,), (0,)), ((), ())),
        preferred_element_type=jnp.float32) + bg_ref[...]
    ids = jax.lax.broadcasted_iota(jnp.int32, logits.shape, 1)
    v0 = jnp.max(logits, axis=1, keepdims=True)
    i0 = jnp.min(jnp.where(logits == v0, ids, n_exp), axis=1, keepdims=True)
    masked = jnp.where(ids == i0, -jnp.inf, logits)
    v1 = jnp.max(masked, axis=1, keepdims=True)
    i1 = jnp.min(jnp.where(masked == v1, ids, n_exp), axis=1, keepdims=True)
    e1 = jnp.exp(v1 - v0)
    g0 = 1.0 / (1.0 + e1)
    g1 = e1 / (1.0 + e1)

    xb = x.astype(jnp.bfloat16)

    b1_all = b1_ref[...]
    b2_all = b2_ref[...]
    acc = jnp.zeros((t, b2_all.shape[1]), jnp.float32)
    for e in range(n_exp):
        w = g0 * (i0 == e).astype(jnp.float32) + g1 * (i1 == e).astype(jnp.float32)
        h = jax.lax.dot_general(
            xb, w1_ref[e], (((1,), (0,)), ((), ())),
            preferred_element_type=jnp.float32) + b1_all[e:e + 1, :]
        hb = jax.nn.silu(h).astype(jnp.bfloat16)
        o = jax.lax.dot_general(
            hb, w2_ref[e], (((1,), (0,)), ((), ())),
            preferred_element_type=jnp.float32) + b2_all[e:e + 1, :]
        acc = acc + w * o
    pi_ref[...] = acc

    v = jax.lax.dot_general(
        xb, wv1_ref[...], (((1,), (0,)), ((), ())),
        preferred_element_type=jnp.float32) + bv1_ref[...]
    vb = jax.nn.silu(v).astype(jnp.bfloat16)
    vf = jax.lax.dot_general(
        vb, wv2_ref[...], (((1,), (0,)), ((), ())),
        preferred_element_type=jnp.float32) + bv2_ref[...]
    vf_ref[...] = jax.nn.silu(vf)


def kernel(features, Wg, bg, W1, b1, W2, b2, Wv1, bv1, Wv2, bv2):
    n, d = features.shape
    e, _, h = W1.shape
    a = W2.shape[2]
    vh1 = Wv1.shape[1]
    vh2 = Wv2.shape[1]
    t = 512 if n % 512 == 0 else n

    w1b = W1.astype(jnp.bfloat16)
    w2b = W2.astype(jnp.bfloat16)
    wv1b = Wv1.astype(jnp.bfloat16)
    wv2b = Wv2.astype(jnp.bfloat16)

    grid = (n // t,)
    full = lambda *shape: pl.BlockSpec(shape, lambda i: (0,) * len(shape))
    out = pl.pallas_call(
        functools.partial(_fused_kernel, n_exp=e),
        grid=grid,
        in_specs=[
            pl.BlockSpec((t, d), lambda i: (i, 0)),
            full(d, e),
            full(1, e),
            full(e, d, h),
            full(e, h),
            full(e, h, a),
            full(e, a),
            full(d, vh1),
            full(1, vh1),
            full(vh1, vh2),
            full(1, vh2),
        ],
        out_specs=[
            pl.BlockSpec((t, a), lambda i: (i, 0)),
            pl.BlockSpec((t, vh2), lambda i: (i, 0)),
        ],
        out_shape=[
            jax.ShapeDtypeStruct((n, a), jnp.float32),
            jax.ShapeDtypeStruct((n, vh2), jnp.float32),
        ],
        compiler_params=pltpu.CompilerParams(
            dimension_semantics=("arbitrary",)),
    )(features, Wg, bg.reshape(1, e), w1b, b1, w2b, b2,
      wv1b, bv1.reshape(1, vh1), wv2b, bv2.reshape(1, vh2))
    return (out[0], out[1])


# speedup vs baseline: 2.1442x; 1.4290x over previous
import functools

import jax
import jax.numpy as jnp
from jax.experimental import pallas as pl
from jax.experimental.pallas import tpu as pltpu


def _fused_kernel(x_ref, wg_ref, bg_ref, w1_ref, b1_ref, w2_ref, b2_ref,
                  wv1_ref, bv1_ref, wv2_ref, bv2_ref, pi_ref, vf_ref,
                  *, n_exp):
    x = x_ref[...]
    t = x.shape[0]

    logits = jax.lax.dot_general(
        x, wg_ref[...], (((1,), (0,)), ((), ())),
        preferred_element_type=jnp.float32) + bg_ref[...]
    ids = jax.lax.broadcasted_iota(jnp.int32, logits.shape, 1)
    v0 = jnp.max(logits, axis=1, keepdims=True)
    i0 = jnp.min(jnp.where(logits == v0, ids, n_exp), axis=1, keepdims=True)
    masked = jnp.where(ids == i0, -jnp.inf, logits)
    v1 = jnp.max(masked, axis=1, keepdims=True)
    i1 = jnp.min(jnp.where(masked == v1, ids, n_exp), axis=1, keepdims=True)
    e1 = jnp.exp(v1 - v0)
    g0 = 1.0 / (1.0 + e1)
    g1 = e1 / (1.0 + e1)

    xb = x.astype(jnp.bfloat16)

    eh = w1_ref.shape[1]
    hdim = eh // n_exp
    h_all = jax.lax.dot_general(
        xb, w1_ref[...], (((1,), (0,)), ((), ())),
        preferred_element_type=jnp.float32) + b1_ref[...]
    lane_e = jax.lax.broadcasted_iota(jnp.int32, (t, eh), 1) // hdim
    w_big = (g0 * (lane_e == i0).astype(jnp.float32)
             + g1 * (lane_e == i1).astype(jnp.float32))
    hw = (jax.nn.silu(h_all) * w_big).astype(jnp.bfloat16)
    pi = jax.lax.dot_general(
        hw, w2_ref[...], (((1,), (0,)), ((), ())),
        preferred_element_type=jnp.float32)
    w_mat = (g0 * (ids == i0).astype(jnp.float32)
             + g1 * (ids == i1).astype(jnp.float32))
    pi_ref[...] = pi + jax.lax.dot_general(
        w_mat, b2_ref[...], (((1,), (0,)), ((), ())),
        preferred_element_type=jnp.float32)

    v = jax.lax.dot_general(
        xb, wv1_ref[...], (((1,), (0,)), ((), ())),
        preferred_element_type=jnp.float32) + bv1_ref[...]
    vb = jax.nn.silu(v).astype(jnp.bfloat16)
    vf = jax.lax.dot_general(
        vb, wv2_ref[...], (((1,), (0,)), ((), ())),
        preferred_element_type=jnp.float32) + bv2_ref[...]
    vf_ref[...] = jax.nn.silu(vf)


def kernel(features, Wg, bg, W1, b1, W2, b2, Wv1, bv1, Wv2, bv2):
    n, d = features.shape
    e, _, h = W1.shape
    a = W2.shape[2]
    vh1 = Wv1.shape[1]
    vh2 = Wv2.shape[1]
    t = 512 if n % 512 == 0 else n

    w1b = W1.transpose(1, 0, 2).reshape(d, e * h).astype(jnp.bfloat16)
    w2b = W2.reshape(e * h, a).astype(jnp.bfloat16)
    b1f = b1.reshape(1, e * h)
    wv1b = Wv1.astype(jnp.bfloat16)
    wv2b = Wv2.astype(jnp.bfloat16)

    grid = (n // t,)
    full = lambda *shape: pl.BlockSpec(shape, lambda i: (0,) * len(shape))
    out = pl.pallas_call(
        functools.partial(_fused_kernel, n_exp=e),
        grid=grid,
        in_specs=[
            pl.BlockSpec((t, d), lambda i: (i, 0)),
            full(d, e),
            full(1, e),
            full(d, e * h),
            full(1, e * h),
            full(e * h, a),
            full(e, a),
            full(d, vh1),
            full(1, vh1),
            full(vh1, vh2),
            full(1, vh2),
        ],
        out_specs=[
            pl.BlockSpec((t, a), lambda i: (i, 0)),
            pl.BlockSpec((t, vh2), lambda i: (i, 0)),
        ],
        out_shape=[
            jax.ShapeDtypeStruct((n, a), jnp.float32),
            jax.ShapeDtypeStruct((n, vh2), jnp.float32),
        ],
        compiler_params=pltpu.CompilerParams(
            dimension_semantics=("arbitrary",)),
    )(features, Wg, bg.reshape(1, e), w1b, b1f, w2b, b2,
      wv1b, bv1.reshape(1, vh1), wv2b, bv2.reshape(1, vh2))
    return (out[0], out[1])


# speedup vs baseline: 2.3548x; 1.0982x over previous
import functools

import jax
import jax.numpy as jnp
from jax.experimental import pallas as pl
from jax.experimental.pallas import tpu as pltpu


def _fused_kernel(x_ref, wg_ref, bg_ref, w1_ref, b1_ref, w2_ref, b2_ref,
                  wv1_ref, bv1_ref, wv2_ref, bv2_ref, pi_ref, vf_ref,
                  *, n_exp):
    x = x_ref[...]
    t = x.shape[0]

    logits = jax.lax.dot_general(
        x, wg_ref[...], (((1,), (0,)), ((), ())),
        preferred_element_type=jnp.float32) + bg_ref[...]
    ids = jax.lax.broadcasted_iota(jnp.int32, logits.shape, 1)
    v0 = jnp.max(logits, axis=1, keepdims=True)
    i0 = jnp.min(jnp.where(logits == v0, ids, n_exp), axis=1, keepdims=True)
    masked = jnp.where(ids == i0, -jnp.inf, logits)
    v1 = jnp.max(masked, axis=1, keepdims=True)
    i1 = jnp.min(jnp.where(masked == v1, ids, n_exp), axis=1, keepdims=True)
    e1 = jnp.exp(v1 - v0)
    g0 = 1.0 / (1.0 + e1)
    g1 = e1 / (1.0 + e1)
    w_mat = (g0 * (ids == i0).astype(jnp.float32)
             + g1 * (ids == i1).astype(jnp.float32))

    xb = x.astype(jnp.bfloat16)

    h_all = jax.lax.dot_general(
        xb, w1_ref[...], (((1,), (0,)), ((), ())),
        preferred_element_type=jnp.float32) + b1_ref[...]
    hb = h_all.astype(jnp.bfloat16)
    half = jnp.bfloat16(0.5)
    s = hb * (half + half * jnp.tanh(hb * half))
    hdim = w1_ref.shape[1] // n_exp
    b2_all = b2_ref[...]
    acc = jnp.zeros((t, b2_all.shape[1]), jnp.float32)
    for e in range(n_exp):
        o = jax.lax.dot_general(
            s[:, e * hdim:(e + 1) * hdim], w2_ref[e * hdim:(e + 1) * hdim, :],
            (((1,), (0,)), ((), ())),
            preferred_element_type=jnp.float32) + b2_all[e:e + 1, :]
        acc = acc + w_mat[:, e:e + 1] * o
    pi_ref[...] = acc

    v = jax.lax.dot_general(
        xb, wv1_ref[...], (((1,), (0,)), ((), ())),
        preferred_element_type=jnp.float32) + bv1_ref[...]
    vb = v.astype(jnp.bfloat16)
    vb = vb * (half + half * jnp.tanh(vb * half))
    vf = jax.lax.dot_general(
        vb, wv2_ref[...], (((1,), (0,)), ((), ())),
        preferred_element_type=jnp.float32) + bv2_ref[...]
    vf_ref[...] = vf * (0.5 + 0.5 * jnp.tanh(vf * 0.5))


def kernel(features, Wg, bg, W1, b1, W2, b2, Wv1, bv1, Wv2, bv2):
    n, d = features.shape
    e, _, h = W1.shape
    a = W2.shape[2]
    vh1 = Wv1.shape[1]
    vh2 = Wv2.shape[1]
    t = 512 if n % 512 == 0 else n

    w1b = W1.transpose(1, 0, 2).reshape(d, e * h).astype(jnp.bfloat16)
    w2b = W2.reshape(e * h, a).astype(jnp.bfloat16)
    b1f = b1.reshape(1, e * h)
    wv1b = Wv1.astype(jnp.bfloat16)
    wv2b = Wv2.astype(jnp.bfloat16)

    grid = (n // t,)
    full = lambda *shape: pl.BlockSpec(shape, lambda i: (0,) * len(shape))
    out = pl.pallas_call(
        functools.partial(_fused_kernel, n_exp=e),
        grid=grid,
        in_specs=[
            pl.BlockSpec((t, d), lambda i: (i, 0)),
            full(d, e),
            full(1, e),
            full(d, e * h),
            full(1, e * h),
            full(e * h, a),
            full(e, a),
            full(d, vh1),
            full(1, vh1),
            full(vh1, vh2),
            full(1, vh2),
        ],
        out_specs=[
            pl.BlockSpec((t, a), lambda i: (i, 0)),
            pl.BlockSpec((t, vh2), lambda i: (i, 0)),
        ],
        out_shape=[
            jax.ShapeDtypeStruct((n, a), jnp.float32),
            jax.ShapeDtypeStruct((n, vh2), jnp.float32),
        ],
        compiler_params=pltpu.CompilerParams(
            dimension_semantics=("arbitrary",)),
    )(features, Wg, bg.reshape(1, e), w1b, b1f, w2b, b2,
      wv1b, bv1.reshape(1, vh1), wv2b, bv2.reshape(1, vh2))
    return (out[0], out[1])


# speedup vs baseline: 2.4379x; 1.0353x over previous
import functools

import jax
import jax.numpy as jnp
from jax.experimental import pallas as pl
from jax.experimental.pallas import tpu as pltpu


def _fused_kernel(x_ref, wg_ref, bg_ref, w1_ref, b1_ref, w2_ref, b2_ref,
                  wv1_ref, bv1_ref, wv2_ref, bv2_ref, pi_ref, vf_ref,
                  *, n_exp):
    x = x_ref[...]
    t = x.shape[0]

    logits = jax.lax.dot_general(
        x, wg_ref[...], (((1,), (0,)), ((), ())),
        preferred_element_type=jnp.float32) + bg_ref[...]
    ids = jax.lax.broadcasted_iota(jnp.int32, logits.shape, 1)
    v0 = jnp.max(logits, axis=1, keepdims=True)
    i0 = jnp.min(jnp.where(logits == v0, ids, n_exp), axis=1, keepdims=True)
    masked = jnp.where(ids == i0, -jnp.inf, logits)
    v1 = jnp.max(masked, axis=1, keepdims=True)
    i1 = jnp.min(jnp.where(masked == v1, ids, n_exp), axis=1, keepdims=True)
    e1 = jnp.exp(v1 - v0)
    g0 = 1.0 / (1.0 + e1)
    g1 = e1 / (1.0 + e1)
    w_mat = (g0 * (ids == i0).astype(jnp.float32)
             + g1 * (ids == i1).astype(jnp.float32))

    xb = x.astype(jnp.bfloat16)

    h_all = jax.lax.dot_general(
        xb, w1_ref[...], (((1,), (0,)), ((), ())),
        preferred_element_type=jnp.float32) + b1_ref[...]
    hb = h_all.astype(jnp.bfloat16)
    half = jnp.bfloat16(0.5)
    s = hb * (half + half * jnp.tanh(hb * half))
    hdim = w1_ref.shape[1] // n_exp
    b2_all = b2_ref[...]
    acc = jnp.zeros((t, b2_all.shape[1]), jnp.float32)
    for e in range(n_exp):
        o = jax.lax.dot_general(
            s[:, e * hdim:(e + 1) * hdim], w2_ref[e * hdim:(e + 1) * hdim, :],
            (((1,), (0,)), ((), ())),
            preferred_element_type=jnp.float32) + b2_all[e:e + 1, :]
        acc = acc + w_mat[:, e:e + 1] * o
    pi_ref[...] = acc

    v = jax.lax.dot_general(
        xb, wv1_ref[...], (((1,), (0,)), ((), ())),
        preferred_element_type=jnp.float32) + bv1_ref[...]
    vb = v.astype(jnp.bfloat16)
    vb = vb * (half + half * jnp.tanh(vb * half))
    vf = jax.lax.dot_general(
        vb, wv2_ref[...], (((1,), (0,)), ((), ())),
        preferred_element_type=jnp.float32) + bv2_ref[...]
    vf_ref[...] = vf * (0.5 + 0.5 * jnp.tanh(vf * 0.5))


def kernel(features, Wg, bg, W1, b1, W2, b2, Wv1, bv1, Wv2, bv2):
    n, d = features.shape
    e, _, h = W1.shape
    a = W2.shape[2]
    vh1 = Wv1.shape[1]
    vh2 = Wv2.shape[1]
    t = 1024 if n % 1024 == 0 else n

    w1b = W1.transpose(1, 0, 2).reshape(d, e * h).astype(jnp.bfloat16)
    w2b = W2.reshape(e * h, a).astype(jnp.bfloat16)
    b1f = b1.reshape(1, e * h)
    wv1b = Wv1.astype(jnp.bfloat16)
    wv2b = Wv2.astype(jnp.bfloat16)

    grid = (n // t,)
    full = lambda *shape: pl.BlockSpec(shape, lambda i: (0,) * len(shape))
    out = pl.pallas_call(
        functools.partial(_fused_kernel, n_exp=e),
        grid=grid,
        in_specs=[
            pl.BlockSpec((t, d), lambda i: (i, 0)),
            full(d, e),
            full(1, e),
            full(d, e * h),
            full(1, e * h),
            full(e * h, a),
            full(e, a),
            full(d, vh1),
            full(1, vh1),
            full(vh1, vh2),
            full(1, vh2),
        ],
        out_specs=[
            pl.BlockSpec((t, a), lambda i: (i, 0)),
            pl.BlockSpec((t, vh2), lambda i: (i, 0)),
        ],
        out_shape=[
            jax.ShapeDtypeStruct((n, a), jnp.float32),
            jax.ShapeDtypeStruct((n, vh2), jnp.float32),
        ],
        compiler_params=pltpu.CompilerParams(
            dimension_semantics=("arbitrary",)),
    )(features, Wg, bg.reshape(1, e), w1b, b1f, w2b, b2,
      wv1b, bv1.reshape(1, vh1), wv2b, bv2.reshape(1, vh2))
    return (out[0], out[1])
